# Initial kernel scaffold; baseline (speedup 1.0000x reference)
import jax
import jax.numpy as jnp
from jax.experimental import pallas as pl


def kernel(posit_embedding, seq_length):
    raise NotImplementedError("write your pallas kernel here")


# speedup vs baseline: 1.2223x; 1.2223x over previous
import functools

import jax
import jax.numpy as jnp
from jax import lax
from jax.experimental import pallas as pl
from jax.experimental.pallas import tpu as pltpu
from jax.experimental.pallas import tpu_sc as plsc

_S = 8192
_D = 128
_L = 16
_NC = 2
_NS = 16
_NW = _NC * _NS
_ROWS_PER_W = _S // _NW
_CHUNK = 128
_NCHUNK = _ROWS_PER_W // _CHUNK

_mesh = plsc.VectorSubcoreMesh(core_axis_name="c", subcore_axis_name="s")


@functools.partial(
    pl.kernel,
    out_type=jax.ShapeDtypeStruct((_S, _D), jnp.float32),
    mesh=_mesh,
    scratch_types=[
        pltpu.VMEM((_L,), jnp.int32),
        pltpu.VMEM((_NCHUNK, _CHUNK), jnp.int32),
        pltpu.VMEM((_CHUNK, _D), jnp.float32),
        pltpu.SemaphoreType.DMA,
    ],
)
def _posit_embed_sc(limit_hbm, table_hbm, out_hbm, limit_v, idx_v, rows_v, sem):
    wid = lax.axis_index("s") * _NC + lax.axis_index("c")
    base = wid * _ROWS_PER_W

    pltpu.sync_copy(limit_hbm, limit_v)
    limit = limit_v[...]

    for j in range(_NCHUNK):
        for i in range(_CHUNK // _L):
            vec = base + (j * _CHUNK + i * _L) + lax.iota(jnp.int32, _L)
            idx_v[j, pl.ds(i * _L, _L)] = jnp.minimum(vec, limit)

    for j in range(_NCHUNK):
        pltpu.async_copy(table_hbm.at[idx_v.at[j]], rows_v, sem).wait()
        pltpu.sync_copy(rows_v, out_hbm.at[pl.ds(base + j * _CHUNK, _CHUNK)])


def kernel(posit_embedding, seq_length):
    limit = jnp.full((_L,), 0, jnp.int32) + (
        jnp.asarray(seq_length, jnp.int32) - 1)
    out = _posit_embed_sc(limit, posit_embedding)
    return out[None]


# speedup vs baseline: 1.2704x; 1.0394x over previous
import functools

import jax
import jax.numpy as jnp
from jax import lax
from jax.experimental import pallas as pl
from jax.experimental.pallas import tpu as pltpu
from jax.experimental.pallas import tpu_sc as plsc

_S = 8192
_D = 128
_L = 16
_NC = 2
_NS = 16
_NW = _NC * _NS
_ROWS_PER_W = _S // _NW
_CHUNK = 128
_NCHUNK = _ROWS_PER_W // _CHUNK

_mesh = plsc.VectorSubcoreMesh(core_axis_name="c", subcore_axis_name="s")


@functools.partial(
    pl.kernel,
    out_type=jax.ShapeDtypeStruct((_S, _D), jnp.float32),
    mesh=_mesh,
    scratch_types=[
        pltpu.VMEM((_L,), jnp.int32),
        pltpu.VMEM((_NCHUNK, _CHUNK), jnp.int32),
        pltpu.VMEM((_NCHUNK, _CHUNK, _D), jnp.float32),
        pltpu.SemaphoreType.DMA,
        pltpu.SemaphoreType.DMA,
        pltpu.SemaphoreType.DMA,
        pltpu.SemaphoreType.DMA,
    ],
)
def _posit_embed_sc(limit_hbm, table_hbm, out_hbm, limit_v, idx_v, rows_v,
                    g0, g1, w0, w1):
    gsems = (g0, g1)
    wsems = (w0, w1)
    wid = lax.axis_index("s") * _NC + lax.axis_index("c")
    base = wid * _ROWS_PER_W

    pltpu.sync_copy(limit_hbm, limit_v)
    limit = limit_v[...]

    gathers = []
    for j in range(_NCHUNK):
        for i in range(_CHUNK // _L):
            vec = base + (j * _CHUNK + i * _L) + lax.iota(jnp.int32, _L)
            idx_v[j, pl.ds(i * _L, _L)] = jnp.minimum(vec, limit)
        gathers.append(
            pltpu.async_copy(table_hbm.at[idx_v.at[j]], rows_v.at[j], gsems[j]))

    writes = []
    for j in range(_NCHUNK):
        gathers[j].wait()
        writes.append(
            pltpu.async_copy(rows_v.at[j],
                             out_hbm.at[pl.ds(base + j * _CHUNK, _CHUNK)],
                             wsems[j]))
    for w in writes:
        w.wait()


def kernel(posit_embedding, seq_length):
    limit = jnp.full((_L,), 0, jnp.int32) + (
        jnp.asarray(seq_length, jnp.int32) - 1)
    out = _posit_embed_sc(limit, posit_embedding)
    return out[None]
